# Initial kernel scaffold; baseline (speedup 1.0000x reference)
import jax
import jax.numpy as jnp
from jax.experimental import pallas as pl


def kernel(x, adj, current_samples):
    raise NotImplementedError("write your pallas kernel here")


# speedup vs baseline: 5.4764x; 5.4764x over previous
import functools

import jax
import jax.numpy as jnp
from jax.experimental import pallas as pl

N_NODES = 2048
D_FEAT = 128
K_SAMPLES = 16
BLK = 256


def _mask_kernel(cs_ref, m_ref):
    cs = cs_ref[...]
    iota = jax.lax.broadcasted_iota(jnp.int32, (BLK, N_NODES), 1)
    m = iota == cs[:, 0:1]
    for s in range(1, K_SAMPLES):
        m = m | (iota == cs[:, s : s + 1])
    m_ref[...] = m.astype(jnp.bfloat16)


def _select_kernel(cs_ref, x_ref, xt_ref, adj_ref, m_ref, out_ref):
    cs = cs_ref[...]
    xb = x_ref[...]
    xt = xt_ref[...]
    adj = adj_ref[...]
    m_all = m_ref[...]

    iota = jax.lax.broadcasted_iota(jnp.int32, (BLK, N_NODES), 1)

    sims = jnp.dot(xb, xt, preferred_element_type=jnp.float32)
    cnt = jnp.dot(adj.astype(jnp.bfloat16), m_all,
                  preferred_element_type=jnp.float32)

    mrows = iota == cs[:, 0:1]
    for s in range(1, K_SAMPLES):
        mrows = mrows | (iota == cs[:, s : s + 1])
    present = (cnt > 0.0) | mrows

    neg = jnp.float32(-jnp.inf)
    masked = jnp.where(present, sims, neg)

    top_cols = []
    for _ in range(K_SAMPLES):
        mx = jnp.max(masked, axis=1, keepdims=True)
        idx = jnp.min(jnp.where(masked == mx, iota, N_NODES), axis=1,
                      keepdims=True)
        top_cols.append(idx)
        masked = jnp.where(iota == idx, neg, masked)
    topk_idx = jnp.concatenate(top_cols, axis=1)

    pool = jnp.sum(present.astype(jnp.int32), axis=1, keepdims=True)

    cur = jnp.where(present, iota, N_NODES)
    small_cols = []
    for _ in range(K_SAMPLES):
        mn = jnp.min(cur, axis=1, keepdims=True)
        small_cols.append(mn)
        cur = jnp.where(cur == mn, N_NODES, cur)
    sorted_vals = jnp.concatenate(small_cols, axis=1)

    jr = jax.lax.broadcasted_iota(jnp.int32, (BLK, K_SAMPLES), 1)
    pad_idx = jnp.clip(jr - pool, 0, K_SAMPLES - 1)
    gath = jnp.where(pad_idx == 0, cs[:, 0:1], 0)
    for kk in range(1, K_SAMPLES):
        gath = gath + jnp.where(pad_idx == kk, cs[:, kk : kk + 1], 0)
    padded = jnp.where(jr < pool, sorted_vals, gath)

    selected = jnp.where(pool >= K_SAMPLES, topk_idx, padded)
    has_nb = jnp.max(adj, axis=1, keepdims=True) > 0.0
    out_ref[...] = jnp.where(has_nb, selected, cs)


def kernel(x, adj, current_samples):
    out_dtype = current_samples.dtype
    cs = current_samples.astype(jnp.int32)

    m = pl.pallas_call(
        _mask_kernel,
        grid=(N_NODES // BLK,),
        in_specs=[pl.BlockSpec((BLK, K_SAMPLES), lambda i: (i, 0))],
        out_specs=pl.BlockSpec((BLK, N_NODES), lambda i: (i, 0)),
        out_shape=jax.ShapeDtypeStruct((N_NODES, N_NODES), jnp.bfloat16),
    )(cs)

    out = pl.pallas_call(
        _select_kernel,
        grid=(N_NODES // BLK,),
        in_specs=[
            pl.BlockSpec((BLK, K_SAMPLES), lambda i: (i, 0)),
            pl.BlockSpec((BLK, D_FEAT), lambda i: (i, 0)),
            pl.BlockSpec((D_FEAT, N_NODES), lambda i: (0, 0)),
            pl.BlockSpec((BLK, N_NODES), lambda i: (i, 0)),
            pl.BlockSpec((N_NODES, N_NODES), lambda i: (0, 0)),
        ],
        out_specs=pl.BlockSpec((BLK, K_SAMPLES), lambda i: (i, 0)),
        out_shape=jax.ShapeDtypeStruct((N_NODES, K_SAMPLES), jnp.int32),
    )(cs, x, x.T, adj, m)

    return out.astype(out_dtype)


# speedup vs baseline: 6.4686x; 1.1812x over previous
import functools

import jax
import jax.numpy as jnp
from jax.experimental import pallas as pl

N_NODES = 2048
D_FEAT = 128
K_SAMPLES = 16
BLK = 256


def _mask_kernel(cs_ref, m_ref):
    cs = cs_ref[...]
    iota = jax.lax.broadcasted_iota(jnp.int32, (BLK, N_NODES), 1)
    m = iota == cs[:, 0:1]
    for s in range(1, K_SAMPLES):
        m = m | (iota == cs[:, s : s + 1])
    m_ref[...] = m.astype(jnp.bfloat16)


def _select_kernel(cs_ref, x_ref, xt_ref, adj_ref, m_ref, out_ref):
    cs = cs_ref[...]
    xb = x_ref[...]
    xt = xt_ref[...]
    adj = adj_ref[...]
    m_all = m_ref[...]

    iota = jax.lax.broadcasted_iota(jnp.int32, (BLK, N_NODES), 1)

    sims = jnp.dot(xb, xt, preferred_element_type=jnp.float32)
    cnt = jnp.dot(adj.astype(jnp.bfloat16), m_all,
                  preferred_element_type=jnp.float32)

    mrows = iota == cs[:, 0:1]
    for s in range(1, K_SAMPLES):
        mrows = mrows | (iota == cs[:, s : s + 1])
    present = (cnt > 0.0) | mrows

    neg = jnp.float32(-jnp.inf)
    masked = jnp.where(present, sims, neg)

    top_cols = []
    for _ in range(K_SAMPLES):
        mx = jnp.max(masked, axis=1, keepdims=True)
        idx = jnp.min(jnp.where(masked == mx, iota, N_NODES), axis=1,
                      keepdims=True)
        top_cols.append(idx)
        masked = jnp.where(iota == idx, neg, masked)
    topk_idx = jnp.concatenate(top_cols, axis=1)

    pool = jnp.sum(present.astype(jnp.int32), axis=1, keepdims=True)
    has_nb = jnp.max(adj, axis=1, keepdims=True) > 0.0

    out_ref[...] = jnp.where(has_nb, topk_idx, cs)

    @pl.when(jnp.any(pool < K_SAMPLES))
    def _fallback():
        cur = jnp.where(present, iota, N_NODES)
        small_cols = []
        for _ in range(K_SAMPLES):
            mn = jnp.min(cur, axis=1, keepdims=True)
            small_cols.append(mn)
            cur = jnp.where(cur == mn, N_NODES, cur)
        sorted_vals = jnp.concatenate(small_cols, axis=1)

        jr = jax.lax.broadcasted_iota(jnp.int32, (BLK, K_SAMPLES), 1)
        pad_idx = jnp.clip(jr - pool, 0, K_SAMPLES - 1)
        gath = jnp.where(pad_idx == 0, cs[:, 0:1], 0)
        for kk in range(1, K_SAMPLES):
            gath = gath + jnp.where(pad_idx == kk, cs[:, kk : kk + 1], 0)
        padded = jnp.where(jr < pool, sorted_vals, gath)

        selected = jnp.where(pool >= K_SAMPLES, topk_idx, padded)
        out_ref[...] = jnp.where(has_nb, selected, cs)


def kernel(x, adj, current_samples):
    out_dtype = current_samples.dtype
    cs = current_samples.astype(jnp.int32)

    m = pl.pallas_call(
        _mask_kernel,
        grid=(N_NODES // BLK,),
        in_specs=[pl.BlockSpec((BLK, K_SAMPLES), lambda i: (i, 0))],
        out_specs=pl.BlockSpec((BLK, N_NODES), lambda i: (i, 0)),
        out_shape=jax.ShapeDtypeStruct((N_NODES, N_NODES), jnp.bfloat16),
    )(cs)

    out = pl.pallas_call(
        _select_kernel,
        grid=(N_NODES // BLK,),
        in_specs=[
            pl.BlockSpec((BLK, K_SAMPLES), lambda i: (i, 0)),
            pl.BlockSpec((BLK, D_FEAT), lambda i: (i, 0)),
            pl.BlockSpec((D_FEAT, N_NODES), lambda i: (0, 0)),
            pl.BlockSpec((BLK, N_NODES), lambda i: (i, 0)),
            pl.BlockSpec((N_NODES, N_NODES), lambda i: (0, 0)),
        ],
        out_specs=pl.BlockSpec((BLK, K_SAMPLES), lambda i: (i, 0)),
        out_shape=jax.ShapeDtypeStruct((N_NODES, K_SAMPLES), jnp.int32),
    )(cs, x, x.T, adj, m)

    return out.astype(out_dtype)


# speedup vs baseline: 7.8701x; 1.2167x over previous
import functools

import jax
import jax.numpy as jnp
from jax.experimental import pallas as pl

N_NODES = 2048
D_FEAT = 128
K_SAMPLES = 16
BLK = 256


def _mask_kernel(cs_ref, m_ref):
    cs = cs_ref[...]
    iota = jax.lax.broadcasted_iota(jnp.int32, (BLK, N_NODES), 1)
    m = iota == cs[:, 0:1]
    for s in range(1, K_SAMPLES):
        m = m | (iota == cs[:, s : s + 1])
    m_ref[...] = m.astype(jnp.bfloat16)


def _select_kernel(cs_ref, x_ref, xt_ref, adj_ref, m_ref, out_ref):
    cs = cs_ref[...]
    xb = x_ref[...]
    xt = xt_ref[...]
    adj = adj_ref[...]
    m_all = m_ref[...]

    iota = jax.lax.broadcasted_iota(jnp.int32, (BLK, N_NODES), 1)

    sims = jnp.dot(xb, xt, preferred_element_type=jnp.float32)
    rowids = (pl.program_id(0) * BLK
              + jax.lax.broadcasted_iota(jnp.int32, (BLK, N_NODES), 0))
    adj_self = adj + jnp.where(iota == rowids, 1.0, 0.0)
    cnt = jnp.dot(adj_self.astype(jnp.bfloat16), m_all,
                  preferred_element_type=jnp.float32)
    present = cnt > 0.0

    neg = jnp.float32(-jnp.inf)
    masked = jnp.where(present, sims, neg)

    top_cols = []
    for _ in range(K_SAMPLES):
        mx = jnp.max(masked, axis=1, keepdims=True)
        idx = jnp.min(jnp.where(masked == mx, iota, N_NODES), axis=1,
                      keepdims=True)
        top_cols.append(idx)
        masked = jnp.where(iota == idx, neg, masked)
    topk_idx = jnp.concatenate(top_cols, axis=1)

    pool = jnp.sum(present.astype(jnp.int32), axis=1, keepdims=True)
    has_nb = jnp.max(adj, axis=1, keepdims=True) > 0.0

    out_ref[...] = jnp.where(has_nb, topk_idx, cs)

    @pl.when(jnp.any(pool < K_SAMPLES))
    def _fallback():
        cur = jnp.where(present, iota, N_NODES)
        small_cols = []
        for _ in range(K_SAMPLES):
            mn = jnp.min(cur, axis=1, keepdims=True)
            small_cols.append(mn)
            cur = jnp.where(cur == mn, N_NODES, cur)
        sorted_vals = jnp.concatenate(small_cols, axis=1)

        jr = jax.lax.broadcasted_iota(jnp.int32, (BLK, K_SAMPLES), 1)
        pad_idx = jnp.clip(jr - pool, 0, K_SAMPLES - 1)
        gath = jnp.where(pad_idx == 0, cs[:, 0:1], 0)
        for kk in range(1, K_SAMPLES):
            gath = gath + jnp.where(pad_idx == kk, cs[:, kk : kk + 1], 0)
        padded = jnp.where(jr < pool, sorted_vals, gath)

        selected = jnp.where(pool >= K_SAMPLES, topk_idx, padded)
        out_ref[...] = jnp.where(has_nb, selected, cs)


def kernel(x, adj, current_samples):
    out_dtype = current_samples.dtype
    cs = current_samples.astype(jnp.int32)

    m = pl.pallas_call(
        _mask_kernel,
        grid=(N_NODES // BLK,),
        in_specs=[pl.BlockSpec((BLK, K_SAMPLES), lambda i: (i, 0))],
        out_specs=pl.BlockSpec((BLK, N_NODES), lambda i: (i, 0)),
        out_shape=jax.ShapeDtypeStruct((N_NODES, N_NODES), jnp.bfloat16),
    )(cs)

    out = pl.pallas_call(
        _select_kernel,
        grid=(N_NODES // BLK,),
        in_specs=[
            pl.BlockSpec((BLK, K_SAMPLES), lambda i: (i, 0)),
            pl.BlockSpec((BLK, D_FEAT), lambda i: (i, 0)),
            pl.BlockSpec((D_FEAT, N_NODES), lambda i: (0, 0)),
            pl.BlockSpec((BLK, N_NODES), lambda i: (i, 0)),
            pl.BlockSpec((N_NODES, N_NODES), lambda i: (0, 0)),
        ],
        out_specs=pl.BlockSpec((BLK, K_SAMPLES), lambda i: (i, 0)),
        out_shape=jax.ShapeDtypeStruct((N_NODES, K_SAMPLES), jnp.int32),
    )(cs, x, x.T, adj, m)

    return out.astype(out_dtype)
